# baseline (speedup 1.0000x reference)
import functools

import jax
import jax.numpy as jnp
from jax import lax
from jax.experimental import pallas as pl
from jax.experimental.pallas import tpu as pltpu
from jax.experimental.pallas import tpu_sc as plsc

_N = 8192
_C = 768
_G = 2
_V = 320
_GV = _G * _V
_VD = 384
_BR = 2048
_NEG = -1e30


def _tc_body(x_ref, wt_ref, b_ref, u_ref, idx0_ref, idx1_ref, cp_ref, pp_ref,
             hist_ref, probs_ref):
    step = pl.program_id(0)
    l = jnp.dot(x_ref[0], wt_ref[...],
                preferred_element_type=jnp.float32) + b_ref[...]
    li = lax.broadcasted_iota(jnp.int32, (1, _GV), 1)
    m0 = li < _V

    max0 = jnp.max(jnp.where(m0, l, _NEG), axis=1, keepdims=True)
    max1 = jnp.max(jnp.where(m0, _NEG, l), axis=1, keepdims=True)
    mx = jnp.where(m0, max0, max1)
    e = jnp.exp(l - mx)
    s0 = jnp.sum(jnp.where(m0, e, 0.0), axis=1, keepdims=True)
    s1 = jnp.sum(jnp.where(m0, 0.0, e), axis=1, keepdims=True)
    p = e * jnp.where(m0, 1.0 / s0, 1.0 / s1)
    psum = jnp.sum(p, axis=0, keepdims=True)

    ismax = l == mx
    enc = jnp.where(ismax, _GV - li, 0)
    a0 = _GV - jnp.max(jnp.where(m0, enc, 0), axis=1, keepdims=True)
    a1 = _GV - jnp.max(jnp.where(m0, 0, enc), axis=1, keepdims=True)
    oh = ((li == a0) | (li == a1)).astype(jnp.float32)
    hsum = jnp.sum(oh, axis=0, keepdims=True)

    t = l - jnp.log(-jnp.log(u_ref[...]))
    t0 = jnp.max(jnp.where(m0, t, _NEG), axis=1, keepdims=True)
    t1 = jnp.max(jnp.where(m0, _NEG, t), axis=1, keepdims=True)
    tmx = jnp.where(m0, t0, t1)
    tenc = jnp.where(t == tmx, _GV - li, 0)
    i0 = _GV - jnp.max(jnp.where(m0, tenc, 0), axis=1, keepdims=True)
    i1 = _GV - jnp.max(jnp.where(m0, 0, tenc), axis=1, keepdims=True)
    idx0_ref[...] = i0.reshape(_BR)
    idx1_ref[...] = i1.reshape(_BR)

    @pl.when(step == 0)
    def _():
        hist_ref[...] = hsum
        probs_ref[...] = psum

    @pl.when(step > 0)
    def _():
        hist_ref[...] += hsum
        probs_ref[...] += psum

    @pl.when(step == pl.num_programs(0) - 1)
    def _():
        n = jnp.float32(_N)
        hp = hist_ref[...] / n
        th = hp * jnp.log(hp + 1e-7)
        cp = (jnp.exp(-jnp.sum(jnp.where(m0, th, 0.0)))
              + jnp.exp(-jnp.sum(jnp.where(m0, 0.0, th))))
        cp_ref[...] = jnp.reshape(cp, (1, 1))
        ap = probs_ref[...] / n
        tp = ap * jnp.log(ap + 1e-7)
        pp = (jnp.exp(-jnp.sum(jnp.where(m0, tp, 0.0)))
              + jnp.exp(-jnp.sum(jnp.where(m0, 0.0, tp))))
        pp_ref[...] = jnp.reshape(pp, (1, 1))


def _tc_stage(x3, wt, b2, u2):
    tpb = x3.shape[1] // _BR
    return pl.pallas_call(
        _tc_body,
        grid=(_N // _BR,),
        in_specs=[
            pl.BlockSpec((1, _BR, _C), lambda i: (i // tpb, i % tpb, 0)),
            pl.BlockSpec((_C, _GV), lambda i: (0, 0)),
            pl.BlockSpec((1, _GV), lambda i: (0, 0)),
            pl.BlockSpec((_BR, _GV), lambda i: (i, 0)),
        ],
        out_specs=[
            pl.BlockSpec((_BR,), lambda i: (i,)),
            pl.BlockSpec((_BR,), lambda i: (i,)),
            pl.BlockSpec((1, 1), lambda i: (0, 0)),
            pl.BlockSpec((1, 1), lambda i: (0, 0)),
        ],
        out_shape=[
            jax.ShapeDtypeStruct((_N,), jnp.int32),
            jax.ShapeDtypeStruct((_N,), jnp.int32),
            jax.ShapeDtypeStruct((1, 1), jnp.float32),
            jax.ShapeDtypeStruct((1, 1), jnp.float32),
        ],
        scratch_shapes=[
            pltpu.VMEM((1, _GV), jnp.float32),
            pltpu.VMEM((1, _GV), jnp.float32),
        ],
    )(x3, wt, b2, u2)


_CHUNK = 128


def _make_sc_gather():
    info = plsc.get_sparse_core_info()
    nw = info.num_cores * info.num_subcores
    per_w = _N // nw
    nchunk = per_w // _CHUNK
    mesh = plsc.VectorSubcoreMesh(core_axis_name="c", subcore_axis_name="s")

    @functools.partial(
        pl.kernel,
        mesh=mesh,
        out_type=jax.ShapeDtypeStruct((_N, _G * _VD), jnp.float32),
        scratch_types=[
            pltpu.VMEM((_CHUNK,), jnp.int32),
            pltpu.VMEM((_CHUNK,), jnp.int32),
            pltpu.VMEM((_CHUNK, _VD), jnp.float32),
            pltpu.VMEM((_CHUNK, _VD), jnp.float32),
            pltpu.SemaphoreType.DMA,
            pltpu.SemaphoreType.DMA,
            pltpu.SemaphoreType.DMA,
            pltpu.SemaphoreType.DMA,
        ],
    )
    def sc_gather(table_hbm, idx0_hbm, idx1_hbm, out_hbm,
                  ia_v, ib_v, ra_v, rb_v, gsa, gsb, wsa, wsb):
        wid = lax.axis_index("s") * info.num_cores + lax.axis_index("c")
        base = wid * per_w
        ops = [(base + j * _CHUNK, g)
               for j in range(nchunk) for g in range(_G)]
        idxs = (idx0_hbm, idx1_hbm)
        idx_v = (ia_v, ib_v)
        rows_v = (ra_v, rb_v)
        gsem = (gsa, gsb)
        wsem = (wsa, wsb)

        def start_gather(k):
            off, g = ops[k]
            p = k % 2
            pltpu.sync_copy(idxs[g].at[pl.ds(off, _CHUNK)], idx_v[p])
            return pltpu.async_copy(table_hbm.at[idx_v[p]], rows_v[p],
                                    gsem[p])

        gh = [start_gather(0), start_gather(1)]
        wh = [None, None]
        for k in range(len(ops)):
            off, g = ops[k]
            p = k % 2
            gh[p].wait()
            wh[p] = pltpu.async_copy(
                rows_v[p],
                out_hbm.at[pl.ds(off, _CHUNK), pl.ds(g * _VD, _VD)],
                wsem[p])
            if k + 2 < len(ops):
                wh[p].wait()
                gh[p] = start_gather(k + 2)
        wh[0].wait()
        wh[1].wait()

    return sc_gather


def kernel(x, W, b, vars_, u):
    bsz, tsz, fsz = x.shape
    wt = W.T
    b2 = b.reshape(1, _GV)
    u2 = u.reshape(_N, _GV)
    idx0, idx1, cp, pp = _tc_stage(x, wt, b2, u2)
    table = vars_.reshape(_GV, _VD)
    outf = _make_sc_gather()(table, idx0, idx1)
    out = outf.reshape(bsz, tsz, _G * _VD)
    return out, cp[0, 0], pp[0, 0]


# speedup vs baseline: 1.0161x; 1.0161x over previous
import functools

import jax
import jax.numpy as jnp
from jax import lax
from jax.experimental import pallas as pl
from jax.experimental.pallas import tpu as pltpu
from jax.experimental.pallas import tpu_sc as plsc

_N = 8192
_C = 768
_G = 2
_V = 320
_GV = _G * _V
_VD = 384
_BR = 1024
_NEG = -1e30


def _tc_body(x_ref, wt_ref, b_ref, u_ref, idx0_ref, idx1_ref, cp_ref, pp_ref,
             hist_ref, probs_ref):
    step = pl.program_id(0)
    l = jnp.dot(x_ref[0], wt_ref[...],
                preferred_element_type=jnp.float32) + b_ref[...]
    li = lax.broadcasted_iota(jnp.int32, (1, _GV), 1)
    m0 = li < _V

    max0 = jnp.max(jnp.where(m0, l, _NEG), axis=1, keepdims=True)
    max1 = jnp.max(jnp.where(m0, _NEG, l), axis=1, keepdims=True)
    mx = jnp.where(m0, max0, max1)
    e = jnp.exp(l - mx)
    s0 = jnp.sum(jnp.where(m0, e, 0.0), axis=1, keepdims=True)
    s1 = jnp.sum(jnp.where(m0, 0.0, e), axis=1, keepdims=True)
    rt0 = jnp.reshape(1.0 / s0, (_BR,))[None, :]
    rt1 = jnp.reshape(1.0 / s1, (_BR,))[None, :]
    psum = jnp.where(m0,
                     jnp.dot(rt0, e, preferred_element_type=jnp.float32),
                     jnp.dot(rt1, e, preferred_element_type=jnp.float32))

    ismax = l == mx
    enc = jnp.where(ismax, _GV - li, 0)
    a0 = _GV - jnp.max(jnp.where(m0, enc, 0), axis=1, keepdims=True)
    a1 = _GV - jnp.max(jnp.where(m0, 0, enc), axis=1, keepdims=True)
    oh = ((li == a0) | (li == a1)).astype(jnp.float32)
    hsum = jnp.dot(jnp.ones((1, _BR), jnp.float32), oh,
                   preferred_element_type=jnp.float32)

    t = l - jnp.log(-jnp.log(u_ref[...]))
    t0 = jnp.max(jnp.where(m0, t, _NEG), axis=1, keepdims=True)
    t1 = jnp.max(jnp.where(m0, _NEG, t), axis=1, keepdims=True)
    tmx = jnp.where(m0, t0, t1)
    tenc = jnp.where(t == tmx, _GV - li, 0)
    i0 = _GV - jnp.max(jnp.where(m0, tenc, 0), axis=1, keepdims=True)
    i1 = _GV - jnp.max(jnp.where(m0, 0, tenc), axis=1, keepdims=True)
    idx0_ref[...] = i0.reshape(_BR)
    idx1_ref[...] = i1.reshape(_BR)

    @pl.when(step == 0)
    def _():
        hist_ref[...] = hsum
        probs_ref[...] = psum

    @pl.when(step > 0)
    def _():
        hist_ref[...] += hsum
        probs_ref[...] += psum

    @pl.when(step == pl.num_programs(0) - 1)
    def _():
        n = jnp.float32(_N)
        hp = hist_ref[...] / n
        th = hp * jnp.log(hp + 1e-7)
        cp = (jnp.exp(-jnp.sum(jnp.where(m0, th, 0.0)))
              + jnp.exp(-jnp.sum(jnp.where(m0, 0.0, th))))
        cp_ref[...] = jnp.reshape(cp, (1, 1))
        ap = probs_ref[...] / n
        tp = ap * jnp.log(ap + 1e-7)
        pp = (jnp.exp(-jnp.sum(jnp.where(m0, tp, 0.0)))
              + jnp.exp(-jnp.sum(jnp.where(m0, 0.0, tp))))
        pp_ref[...] = jnp.reshape(pp, (1, 1))


def _tc_stage(x3, wt, b2, u2):
    tpb = x3.shape[1] // _BR
    return pl.pallas_call(
        _tc_body,
        grid=(_N // _BR,),
        in_specs=[
            pl.BlockSpec((1, _BR, _C), lambda i: (i // tpb, i % tpb, 0)),
            pl.BlockSpec((_C, _GV), lambda i: (0, 0)),
            pl.BlockSpec((1, _GV), lambda i: (0, 0)),
            pl.BlockSpec((_BR, _GV), lambda i: (i, 0)),
        ],
        out_specs=[
            pl.BlockSpec((_BR,), lambda i: (i,)),
            pl.BlockSpec((_BR,), lambda i: (i,)),
            pl.BlockSpec((1, 1), lambda i: (0, 0)),
            pl.BlockSpec((1, 1), lambda i: (0, 0)),
        ],
        out_shape=[
            jax.ShapeDtypeStruct((_N,), jnp.int32),
            jax.ShapeDtypeStruct((_N,), jnp.int32),
            jax.ShapeDtypeStruct((1, 1), jnp.float32),
            jax.ShapeDtypeStruct((1, 1), jnp.float32),
        ],
        scratch_shapes=[
            pltpu.VMEM((1, _GV), jnp.float32),
            pltpu.VMEM((1, _GV), jnp.float32),
        ],
    )(x3, wt, b2, u2)


_CHUNK = 128


def _make_sc_gather():
    info = plsc.get_sparse_core_info()
    nw = info.num_cores * info.num_subcores
    per_w = _N // nw
    nchunk = per_w // _CHUNK
    mesh = plsc.VectorSubcoreMesh(core_axis_name="c", subcore_axis_name="s")

    @functools.partial(
        pl.kernel,
        mesh=mesh,
        out_type=jax.ShapeDtypeStruct((_N, _G * _VD), jnp.float32),
        scratch_types=[
            pltpu.VMEM((_CHUNK,), jnp.int32),
            pltpu.VMEM((_CHUNK,), jnp.int32),
            pltpu.VMEM((_CHUNK, _VD), jnp.float32),
            pltpu.VMEM((_CHUNK, _VD), jnp.float32),
            pltpu.SemaphoreType.DMA,
            pltpu.SemaphoreType.DMA,
            pltpu.SemaphoreType.DMA,
            pltpu.SemaphoreType.DMA,
        ],
    )
    def sc_gather(table_hbm, idx0_hbm, idx1_hbm, out_hbm,
                  ia_v, ib_v, ra_v, rb_v, gsa, gsb, wsa, wsb):
        wid = lax.axis_index("s") * info.num_cores + lax.axis_index("c")
        base = wid * per_w
        ops = [(base + j * _CHUNK, g)
               for j in range(nchunk) for g in range(_G)]
        idxs = (idx0_hbm, idx1_hbm)
        idx_v = (ia_v, ib_v)
        rows_v = (ra_v, rb_v)
        gsem = (gsa, gsb)
        wsem = (wsa, wsb)

        def start_gather(k):
            off, g = ops[k]
            p = k % 2
            pltpu.sync_copy(idxs[g].at[pl.ds(off, _CHUNK)], idx_v[p])
            return pltpu.async_copy(table_hbm.at[idx_v[p]], rows_v[p],
                                    gsem[p])

        gh = [start_gather(0), start_gather(1)]
        wh = [None, None]
        for k in range(len(ops)):
            off, g = ops[k]
            p = k % 2
            gh[p].wait()
            wh[p] = pltpu.async_copy(
                rows_v[p],
                out_hbm.at[pl.ds(off, _CHUNK), pl.ds(g * _VD, _VD)],
                wsem[p])
            if k + 2 < len(ops):
                wh[p].wait()
                gh[p] = start_gather(k + 2)
        wh[0].wait()
        wh[1].wait()

    return sc_gather


def kernel(x, W, b, vars_, u):
    bsz, tsz, fsz = x.shape
    wt = W.T
    b2 = b.reshape(1, _GV)
    u2 = u.reshape(_N, _GV)
    idx0, idx1, cp, pp = _tc_stage(x, wt, b2, u2)
    table = vars_.reshape(_GV, _VD)
    outf = _make_sc_gather()(table, idx0, idx1)
    out = outf.reshape(bsz, tsz, _G * _VD)
    return out, cp[0, 0], pp[0, 0]
